# Initial kernel scaffold; baseline (speedup 1.0000x reference)
import jax
import jax.numpy as jnp
from jax.experimental import pallas as pl


def kernel(action, table, W1, b1, g1, be1, W2, b2, g2, be2):
    raise NotImplementedError("write your pallas kernel here")


# speedup vs baseline: 5.6309x; 5.6309x over previous
import functools

import jax
import jax.numpy as jnp
from jax import lax
from jax.experimental import pallas as pl
from jax.experimental.pallas import tpu as pltpu
from jax.experimental.pallas import tpu_sc as plsc

_EPS = 1e-5

_V, _D = 1000, 64
_B, _L = 16384, 50
_BL = _B * _L

_GROUP = 128
_GPC = 4
_CH = _GPC * _GROUP


def _layer_norm(x, g, b):
    m = jnp.mean(x, axis=-1, keepdims=True)
    v = jnp.mean((x - m) ** 2, axis=-1, keepdims=True)
    return (x - m) * lax.rsqrt(v + _EPS) * g + b


def _transform_body(t_ref, w1_ref, b1_ref, g1_ref, be1_ref,
                    w2_ref, b2_ref, g2_ref, be2_ref, o_ref):
    x = t_ref[...]
    x = jnp.dot(x, w1_ref[...], preferred_element_type=jnp.float32) + b1_ref[...]
    x = _layer_norm(x, g1_ref[...], be1_ref[...])
    x = jnp.maximum(x, 0.0)
    x = jnp.dot(x, w2_ref[...], preferred_element_type=jnp.float32) + b2_ref[...]
    x = _layer_norm(x, g2_ref[...], be2_ref[...])
    o_ref[...] = x


def _transform_table(table, w1t, b1, g1, be1, w2t, b2, g2, be2):
    v, d = table.shape
    row = lambda a: a.reshape(1, d)
    return pl.pallas_call(
        _transform_body,
        out_shape=jax.ShapeDtypeStruct((v, d), jnp.float32),
    )(table, w1t, row(b1), row(g1), row(be1), w2t, row(b2), row(g2), row(be2))


def _make_sc_gather(bl, d):
    info = plsc.get_sparse_core_info()
    nc, ns = info.num_cores, info.num_subcores
    nw = nc * ns
    rows_w = bl // nw
    ng = rows_w // _GROUP
    nchunk = ng // _GPC
    mesh = plsc.VectorSubcoreMesh(core_axis_name="c", subcore_axis_name="s")

    @functools.partial(
        pl.kernel, mesh=mesh,
        out_type=jax.ShapeDtypeStruct((bl, d), jnp.float32),
        compiler_params=pltpu.CompilerParams(use_tc_tiling_on_sc=False),
        scratch_types=[
            pltpu.VMEM((ng, _GROUP), jnp.int32),
            pltpu.VMEM((_CH, d), jnp.float32),
            pltpu.SemaphoreType.DMA,
        ],
    )
    def gather(tab_hbm, idx_hbm, out_hbm, idx_v, rows_v, gsem):
        wid = lax.axis_index("s") * nc + lax.axis_index("c")
        base = wid * rows_w
        pltpu.sync_copy(idx_hbm.at[pl.ds(wid * ng, ng)], idx_v)

        def chunk(c, carry):
            cps = []
            for j in range(_GPC):
                cps.append(pltpu.async_copy(
                    tab_hbm.at[idx_v.at[c * _GPC + j]],
                    rows_v.at[pl.ds(j * _GROUP, _GROUP)],
                    gsem))
            for cp in cps:
                cp.wait()
            pltpu.sync_copy(rows_v, out_hbm.at[pl.ds(base + c * _CH, _CH)])
            return carry

        lax.fori_loop(0, nchunk, chunk, 0)

    return gather


_sc_gather = _make_sc_gather(_BL, _D)


def kernel(action, table, W1, b1, g1, be1, W2, b2, g2, be2):
    tab_t = _transform_table(table, W1.T, b1, g1, be1, W2.T, b2, g2, be2)
    idx = action.reshape(_BL // _GROUP, _GROUP)
    out = _sc_gather(tab_t, idx)
    return out.reshape(_B, _L, _D)


# speedup vs baseline: 5.6402x; 1.0017x over previous
import functools

import jax
import jax.numpy as jnp
from jax import lax
from jax.experimental import pallas as pl
from jax.experimental.pallas import tpu as pltpu
from jax.experimental.pallas import tpu_sc as plsc

_EPS = 1e-5

_V, _D = 1000, 64
_B, _L = 16384, 50
_BL = _B * _L

_CH = 512
_W = 5


def _layer_norm(x, g, b):
    m = jnp.mean(x, axis=-1, keepdims=True)
    v = jnp.mean((x - m) ** 2, axis=-1, keepdims=True)
    return (x - m) * lax.rsqrt(v + _EPS) * g + b


def _transform_body(t_ref, w1_ref, b1_ref, g1_ref, be1_ref,
                    w2_ref, b2_ref, g2_ref, be2_ref, o_ref):
    x = t_ref[...]
    x = jnp.dot(x, w1_ref[...], preferred_element_type=jnp.float32) + b1_ref[...]
    x = _layer_norm(x, g1_ref[...], be1_ref[...])
    x = jnp.maximum(x, 0.0)
    x = jnp.dot(x, w2_ref[...], preferred_element_type=jnp.float32) + b2_ref[...]
    x = _layer_norm(x, g2_ref[...], be2_ref[...])
    o_ref[...] = x


def _transform_table(table, w1t, b1, g1, be1, w2t, b2, g2, be2):
    v, d = table.shape
    row = lambda a: a.reshape(1, d)
    return pl.pallas_call(
        _transform_body,
        out_shape=jax.ShapeDtypeStruct((v, d), jnp.float32),
    )(table, w1t, row(b1), row(g1), row(be1), w2t, row(b2), row(g2), row(be2))


def _make_sc_gather(bl, d):
    info = plsc.get_sparse_core_info()
    nc, ns = info.num_cores, info.num_subcores
    nw = nc * ns
    rows_w = bl // nw
    nchunk = rows_w // _CH
    nwin = nchunk // _W
    mesh = plsc.VectorSubcoreMesh(core_axis_name="c", subcore_axis_name="s")

    @functools.partial(
        pl.kernel, mesh=mesh,
        out_type=jax.ShapeDtypeStruct((bl, d), jnp.float32),
        compiler_params=pltpu.CompilerParams(use_tc_tiling_on_sc=False),
        scratch_types=[
            pltpu.VMEM((rows_w,), jnp.int32),
            pltpu.VMEM((2, _CH, d), jnp.float32),
            pltpu.SemaphoreType.DMA,
        ],
    )
    def gather(tab_hbm, idx_hbm, out_hbm, idx_v, rows_v, gsem):
        wid = lax.axis_index("s") * nc + lax.axis_index("c")
        base = wid * rows_w
        pltpu.sync_copy(idx_hbm.at[pl.ds(base, rows_w)], idx_v)

        def window(w, carry):
            c0 = w * _W
            cps = [pltpu.async_copy(
                tab_hbm.at[idx_v.at[pl.ds(c0 * _CH, _CH)]], rows_v.at[0], gsem)]
            for j in range(_W):
                c = c0 + j
                cps[j].wait()
                if j + 1 < _W:
                    cps.append(pltpu.async_copy(
                        tab_hbm.at[idx_v.at[pl.ds((c0 + j + 1) * _CH, _CH)]],
                        rows_v.at[(j + 1) % 2], gsem))
                pltpu.sync_copy(rows_v.at[j % 2],
                                out_hbm.at[pl.ds(base + c * _CH, _CH)])
            return carry

        lax.fori_loop(0, nwin, window, 0)

    return gather


_sc_gather = _make_sc_gather(_BL, _D)


def kernel(action, table, W1, b1, g1, be1, W2, b2, g2, be2):
    tab_t = _transform_table(table, W1.T, b1, g1, be1, W2.T, b2, g2, be2)
    idx = action.reshape(_BL)
    out = _sc_gather(tab_t, idx)
    return out.reshape(_B, _L, _D)


# speedup vs baseline: 7.7464x; 1.3734x over previous
import functools

import jax
import jax.numpy as jnp
from jax import lax
from jax.experimental import pallas as pl
from jax.experimental.pallas import tpu as pltpu
from jax.experimental.pallas import tpu_sc as plsc

_EPS = 1e-5

_V, _D = 1000, 64
_B, _L = 16384, 50
_BL = _B * _L

_CH = 512
_W = 5


def _layer_norm(x, g, b):
    m = jnp.mean(x, axis=-1, keepdims=True)
    v = jnp.mean((x - m) ** 2, axis=-1, keepdims=True)
    return (x - m) * lax.rsqrt(v + _EPS) * g + b


def _transform_body(t_ref, w1_ref, b1_ref, g1_ref, be1_ref,
                    w2_ref, b2_ref, g2_ref, be2_ref, o_ref):
    x = t_ref[...]
    x = jnp.dot(x, w1_ref[...], preferred_element_type=jnp.float32) + b1_ref[...]
    x = _layer_norm(x, g1_ref[...], be1_ref[...])
    x = jnp.maximum(x, 0.0)
    x = jnp.dot(x, w2_ref[...], preferred_element_type=jnp.float32) + b2_ref[...]
    x = _layer_norm(x, g2_ref[...], be2_ref[...])
    o_ref[...] = x


def _transform_table(table, w1t, b1, g1, be1, w2t, b2, g2, be2):
    v, d = table.shape
    row = lambda a: a.reshape(1, d)
    return pl.pallas_call(
        _transform_body,
        out_shape=jax.ShapeDtypeStruct((v, d), jnp.float32),
    )(table, w1t, row(b1), row(g1), row(be1), w2t, row(b2), row(g2), row(be2))


def _make_sc_gather(bl, d):
    info = plsc.get_sparse_core_info()
    nc, ns = info.num_cores, info.num_subcores
    nw = nc * ns
    rows_w = bl // nw
    nchunk = rows_w // _CH
    nwin = nchunk // _W
    mesh = plsc.VectorSubcoreMesh(core_axis_name="c", subcore_axis_name="s")

    @functools.partial(
        pl.kernel, mesh=mesh,
        out_type=jax.ShapeDtypeStruct((bl, d), jnp.float32),
        compiler_params=pltpu.CompilerParams(use_tc_tiling_on_sc=False),
        scratch_types=[
            pltpu.VMEM((rows_w,), jnp.int32),
            pltpu.VMEM((2, _CH, d), jnp.float32),
            pltpu.VMEM_SHARED((_V, d), jnp.float32),
            pltpu.SemaphoreType.DMA,
        ],
    )
    def gather(tab_hbm, idx_hbm, out_hbm, idx_v, rows_v, tab_sh, gsem):
        sid = lax.axis_index("s")
        wid = sid * nc + lax.axis_index("c")
        base = wid * rows_w
        @pl.when(sid == 0)
        def _():
            pltpu.sync_copy(tab_hbm, tab_sh)
        pltpu.sync_copy(idx_hbm.at[pl.ds(base, rows_w)], idx_v)
        plsc.subcore_barrier()

        def window(w, carry):
            c0 = w * _W
            cps = [pltpu.async_copy(
                tab_sh.at[idx_v.at[pl.ds(c0 * _CH, _CH)]], rows_v.at[0], gsem)]
            for j in range(_W):
                c = c0 + j
                cps[j].wait()
                if j + 1 < _W:
                    cps.append(pltpu.async_copy(
                        tab_sh.at[idx_v.at[pl.ds((c0 + j + 1) * _CH, _CH)]],
                        rows_v.at[(j + 1) % 2], gsem))
                pltpu.sync_copy(rows_v.at[j % 2],
                                out_hbm.at[pl.ds(base + c * _CH, _CH)])
            return carry

        lax.fori_loop(0, nwin, window, 0)

    return gather


_sc_gather = _make_sc_gather(_BL, _D)


def kernel(action, table, W1, b1, g1, be1, W2, b2, g2, be2):
    tab_t = _transform_table(table, W1.T, b1, g1, be1, W2.T, b2, g2, be2)
    idx = action.reshape(_BL)
    out = _sc_gather(tab_t, idx)
    return out.reshape(_B, _L, _D)


# speedup vs baseline: 9.9789x; 1.2882x over previous
import functools

import jax
import jax.numpy as jnp
from jax import lax
from jax.experimental import pallas as pl
from jax.experimental.pallas import tpu as pltpu
from jax.experimental.pallas import tpu_sc as plsc

_EPS = 1e-5

_V, _D = 1000, 64
_B, _L = 16384, 50
_BL = _B * _L
_LP, _DP = 56, 128

_NB = 8
_CH = _NB * _L


def _layer_norm(x, g, b):
    m = jnp.mean(x, axis=-1, keepdims=True)
    v = jnp.mean((x - m) ** 2, axis=-1, keepdims=True)
    return (x - m) * lax.rsqrt(v + _EPS) * g + b


def _transform_body(t_ref, w1_ref, b1_ref, g1_ref, be1_ref,
                    w2_ref, b2_ref, g2_ref, be2_ref, o_ref):
    x = t_ref[...]
    x = jnp.dot(x, w1_ref[...], preferred_element_type=jnp.float32) + b1_ref[...]
    x = _layer_norm(x, g1_ref[...], be1_ref[...])
    x = jnp.maximum(x, 0.0)
    x = jnp.dot(x, w2_ref[...], preferred_element_type=jnp.float32) + b2_ref[...]
    x = _layer_norm(x, g2_ref[...], be2_ref[...])
    o_ref[...] = jnp.concatenate([x, jnp.zeros_like(x)], axis=1)


def _transform_table(table, w1t, b1, g1, be1, w2t, b2, g2, be2):
    v, d = table.shape
    row = lambda a: a.reshape(1, d)
    return pl.pallas_call(
        _transform_body,
        out_shape=jax.ShapeDtypeStruct((v, _DP), jnp.float32),
    )(table, w1t, row(b1), row(g1), row(be1), w2t, row(b2), row(g2), row(be2))


def _make_sc_gather(bb, ll, d):
    info = plsc.get_sparse_core_info()
    nc, ns = info.num_cores, info.num_subcores
    nw = nc * ns
    b_w = bb // nw
    t_w = b_w * ll
    nchunk = b_w // _NB
    mesh = plsc.VectorSubcoreMesh(core_axis_name="c", subcore_axis_name="s")

    @functools.partial(
        pl.kernel, mesh=mesh,
        out_type=jax.ShapeDtypeStruct((bb, _LP, _DP), jnp.float32),
        compiler_params=pltpu.CompilerParams(use_tc_tiling_on_sc=False),
        scratch_types=[
            pltpu.VMEM((t_w,), jnp.int32),
            pltpu.VMEM((_CH, _DP), jnp.float32),
            pltpu.VMEM_SHARED((_V, _DP), jnp.float32),
            pltpu.SemaphoreType.DMA,
        ],
    )
    def gather(tab_hbm, idx_hbm, out_hbm, idx_v, rows_v, tab_sh, gsem):
        sid = lax.axis_index("s")
        wid = sid * nc + lax.axis_index("c")
        base = wid * b_w
        @pl.when(sid == 0)
        def _():
            pltpu.sync_copy(tab_hbm, tab_sh)
        pltpu.sync_copy(idx_hbm.at[pl.ds(wid * t_w, t_w)], idx_v)
        plsc.subcore_barrier()

        def chunk(c, carry):
            cp = pltpu.async_copy(
                tab_sh.at[idx_v.at[pl.ds(c * _CH, _CH)]], rows_v, gsem)
            cp.wait()
            for b in range(_NB):
                pltpu.sync_copy(rows_v.at[pl.ds(b * ll, ll)],
                                out_hbm.at[base + c * _NB + b, pl.ds(0, ll)])
            return carry

        lax.fori_loop(0, nchunk, chunk, 0)

    return gather


_sc_gather = _make_sc_gather(_B, _L, _D)


def kernel(action, table, W1, b1, g1, be1, W2, b2, g2, be2):
    tab_t = _transform_table(table, W1.T, b1, g1, be1, W2.T, b2, g2, be2)
    idx = action.reshape(_BL)
    out_p = _sc_gather(tab_t, idx)
    return out_p[:, :_L, :_D]


# speedup vs baseline: 12.0518x; 1.2077x over previous
import functools

import jax
import jax.numpy as jnp
from jax import lax
from jax.experimental import pallas as pl
from jax.experimental.pallas import tpu as pltpu
from jax.experimental.pallas import tpu_sc as plsc

_EPS = 1e-5

_V, _D = 1000, 64
_B, _L = 16384, 50
_BL = _B * _L
_LP, _DP = 56, 128

_NB = 4
_CH = _NB * _L
_W = 4


def _layer_norm(x, g, b):
    m = jnp.mean(x, axis=-1, keepdims=True)
    v = jnp.mean((x - m) ** 2, axis=-1, keepdims=True)
    return (x - m) * lax.rsqrt(v + _EPS) * g + b


def _transform_body(t_ref, w1_ref, b1_ref, g1_ref, be1_ref,
                    w2_ref, b2_ref, g2_ref, be2_ref, o_ref):
    x = t_ref[...]
    x = jnp.dot(x, w1_ref[...], preferred_element_type=jnp.float32) + b1_ref[...]
    x = _layer_norm(x, g1_ref[...], be1_ref[...])
    x = jnp.maximum(x, 0.0)
    x = jnp.dot(x, w2_ref[...], preferred_element_type=jnp.float32) + b2_ref[...]
    x = _layer_norm(x, g2_ref[...], be2_ref[...])
    o_ref[...] = jnp.concatenate([x, jnp.zeros_like(x)], axis=1)


def _transform_table(table, w1t, b1, g1, be1, w2t, b2, g2, be2):
    v, d = table.shape
    row = lambda a: a.reshape(1, d)
    return pl.pallas_call(
        _transform_body,
        out_shape=jax.ShapeDtypeStruct((v, _DP), jnp.float32),
    )(table, w1t, row(b1), row(g1), row(be1), w2t, row(b2), row(g2), row(be2))


def _make_sc_gather(bb, ll, d):
    info = plsc.get_sparse_core_info()
    nc, ns = info.num_cores, info.num_subcores
    nw = nc * ns
    b_w = bb // nw
    t_w = b_w * ll
    nchunk = b_w // _NB
    nwin = nchunk // _W
    mesh = plsc.VectorSubcoreMesh(core_axis_name="c", subcore_axis_name="s")

    @functools.partial(
        pl.kernel, mesh=mesh,
        out_type=jax.ShapeDtypeStruct((bb, _LP, _DP), jnp.float32),
        compiler_params=pltpu.CompilerParams(use_tc_tiling_on_sc=False),
        scratch_types=[
            pltpu.VMEM((t_w,), jnp.int32),
            pltpu.VMEM((2, _CH, _DP), jnp.float32),
            pltpu.VMEM_SHARED((_V, _DP), jnp.float32),
            pltpu.SemaphoreType.DMA,
        ],
    )
    def gather(tab_hbm, idx_hbm, out_hbm, idx_v, rows_v, tab_sh, gsem):
        sid = lax.axis_index("s")
        wid = sid * nc + lax.axis_index("c")
        base = wid * b_w
        @pl.when(sid == 0)
        def _():
            pltpu.sync_copy(tab_hbm, tab_sh)
        pltpu.sync_copy(idx_hbm.at[pl.ds(wid * t_w, t_w)], idx_v)
        plsc.subcore_barrier()

        def fire(c, slot):
            return pltpu.async_copy(
                tab_sh.at[idx_v.at[pl.ds(c * _CH, _CH)]], rows_v.at[slot], gsem)

        def window(w, carry):
            c0 = w * _W
            cps = [fire(c0, 0)]
            for j in range(_W):
                c = c0 + j
                cps[j].wait()
                if j + 1 < _W:
                    cps.append(fire(c0 + j + 1, (j + 1) % 2))
                for b in range(_NB):
                    pltpu.sync_copy(
                        rows_v.at[j % 2, pl.ds(b * ll, ll), pl.ds(0, d)],
                        out_hbm.at[base + c * _NB + b, pl.ds(0, ll), pl.ds(0, d)])
            return carry

        lax.fori_loop(0, nwin, window, 0)

    return gather


_sc_gather = _make_sc_gather(_B, _L, _D)


def kernel(action, table, W1, b1, g1, be1, W2, b2, g2, be2):
    tab_t = _transform_table(table, W1.T, b1, g1, be1, W2.T, b2, g2, be2)
    idx = action.reshape(_BL)
    out_p = _sc_gather(tab_t, idx)
    return out_p[:, :_L, :_D]


# speedup vs baseline: 12.0660x; 1.0012x over previous
import functools

import jax
import jax.numpy as jnp
from jax import lax
from jax.experimental import pallas as pl
from jax.experimental.pallas import tpu as pltpu
from jax.experimental.pallas import tpu_sc as plsc

_EPS = 1e-5

_V, _D = 1000, 64
_B, _L = 16384, 50
_BL = _B * _L
_LP, _DP = 56, 128

_NB = 4
_CH = _NB * _L
_W = 4


def _layer_norm(x, g, b):
    m = jnp.mean(x, axis=-1, keepdims=True)
    v = jnp.mean((x - m) ** 2, axis=-1, keepdims=True)
    return (x - m) / jnp.sqrt(v + _EPS) * g + b


def _transform_body(t_ref, w1_ref, b1_ref, g1_ref, be1_ref,
                    w2_ref, b2_ref, g2_ref, be2_ref, o_ref):
    x = t_ref[...]
    x = jnp.dot(x, w1_ref[...], preferred_element_type=jnp.float32) + b1_ref[...]
    x = _layer_norm(x, g1_ref[...], be1_ref[...])
    x = jnp.maximum(x, 0.0)
    x = jnp.dot(x, w2_ref[...], preferred_element_type=jnp.float32) + b2_ref[...]
    x = _layer_norm(x, g2_ref[...], be2_ref[...])
    o_ref[...] = jnp.concatenate([x, jnp.zeros_like(x)], axis=1)


def _transform_table(table, w1t, b1, g1, be1, w2t, b2, g2, be2):
    v, d = table.shape
    row = lambda a: a.reshape(1, d)
    return pl.pallas_call(
        _transform_body,
        out_shape=jax.ShapeDtypeStruct((v, _DP), jnp.float32),
    )(table, w1t, row(b1), row(g1), row(be1), w2t, row(b2), row(g2), row(be2))


def _make_sc_gather(bb, ll, d):
    info = plsc.get_sparse_core_info()
    nc, ns = info.num_cores, info.num_subcores
    nw = nc * ns
    b_w = bb // nw
    t_w = b_w * ll
    nchunk = b_w // _NB
    nwin = nchunk // _W
    mesh = plsc.VectorSubcoreMesh(core_axis_name="c", subcore_axis_name="s")

    @functools.partial(
        pl.kernel, mesh=mesh,
        out_type=jax.ShapeDtypeStruct((bb, _LP, _DP), jnp.float32),
        compiler_params=pltpu.CompilerParams(use_tc_tiling_on_sc=False),
        scratch_types=[
            pltpu.VMEM((t_w,), jnp.int32),
            pltpu.VMEM((2, _CH, _DP), jnp.float32),
            pltpu.VMEM_SHARED((_V, _DP), jnp.float32),
            pltpu.SemaphoreType.DMA,
        ],
    )
    def gather(tab_hbm, idx_hbm, out_hbm, idx_v, rows_v, tab_sh, gsem):
        sid = lax.axis_index("s")
        wid = sid * nc + lax.axis_index("c")
        base = wid * b_w
        @pl.when(sid == 0)
        def _():
            pltpu.sync_copy(tab_hbm, tab_sh)
        pltpu.sync_copy(idx_hbm.at[pl.ds(wid * t_w, t_w)], idx_v)
        plsc.subcore_barrier()

        def fire(c, slot):
            return pltpu.async_copy(
                tab_sh.at[idx_v.at[pl.ds(c * _CH, _CH)]], rows_v.at[slot], gsem)

        def window(w, carry):
            c0 = w * _W
            cps = [fire(c0, 0)]
            for j in range(_W):
                c = c0 + j
                cps[j].wait()
                if j + 1 < _W:
                    cps.append(fire(c0 + j + 1, (j + 1) % 2))
                for b in range(_NB):
                    pltpu.sync_copy(
                        rows_v.at[j % 2, pl.ds(b * ll, ll), pl.ds(0, d)],
                        out_hbm.at[base + c * _NB + b, pl.ds(0, ll), pl.ds(0, d)])
            return carry

        lax.fori_loop(0, nwin, window, 0)

    return gather


_sc_gather = _make_sc_gather(_B, _L, _D)


def kernel(action, table, W1, b1, g1, be1, W2, b2, g2, be2):
    tab_t = _transform_table(table, W1.T, b1, g1, be1, W2.T, b2, g2, be2)
    idx = action.reshape(_BL)
    out_p = _sc_gather(tab_t, idx)
    return out_p[:, :_L, :_D]
